# baseline (speedup 1.0000x reference)
import jax
import jax.numpy as jnp
from jax import lax
from jax.experimental import pallas as pl
from jax.experimental.pallas import tpu as pltpu
from jax.experimental.pallas import tpu_sc as plsc

L = 16
N_EXP = 1024
N_STR = 4096
ROWS = 16


def _count_le(grid_v, n, strides, x_vec, iota):
    base = iota * 0
    cnt = base
    for st in strides:
        probe = base + iota * st
        in_range = probe < n
        probe_c = jnp.minimum(probe, n - 1)
        vals = plsc.load_gather(grid_v, [probe_c])
        le = jnp.logical_and(vals <= x_vec, in_range)
        cnt = plsc.all_reduce_population_count(le)
        if st != 1:
            base = base + jnp.maximum(cnt - 1, 0) * st
    return base + cnt


def _body(vols_hbm, strikes_hbm, expiries_hbm, strike_hbm, expiry_hbm,
          out_hbm, strikes_v, expiries_v, strike_v, expiry_v, win_v, out_v,
          sem_e, sem_s, sem_w):
    c = lax.axis_index("c")
    s = lax.axis_index("s")

    @pl.when(jnp.logical_and(c == 0, s == 0))
    def _():
        cp_e = pltpu.async_copy(expiries_hbm, expiries_v, sem_e)
        cp_t = pltpu.async_copy(expiry_hbm, expiry_v, sem_e)
        cp_s = pltpu.async_copy(strikes_hbm, strikes_v, sem_s)
        cp_k = pltpu.async_copy(strike_hbm, strike_v, sem_s)

        iota = lax.iota(jnp.int32, L)
        zeros = iota * 0

        cp_e.wait()
        cp_t.wait()
        t = plsc.load_gather(expiry_v, [zeros])
        n_e = _count_le(expiries_v, N_EXP, [64, 4, 1], t, iota)
        jhi = jnp.clip(n_e, 1, N_EXP - 1)
        jlo = jhi - 1
        rbase = (jlo >> 3) << 3
        rbase_s = pl.multiple_of(jnp.max(rbase), 8)
        cp_w = pltpu.async_copy(vols_hbm.at[pl.ds(rbase_s, ROWS)], win_v,
                                sem_w)

        e_lo = plsc.load_gather(expiries_v, [jlo])
        e_hi = plsc.load_gather(expiries_v, [jhi])
        u = jnp.clip((t - e_lo) / (e_hi - e_lo), 0.0, 1.0)

        cp_s.wait()
        cp_k.wait()
        s_first = plsc.load_gather(strikes_v, [zeros])
        s_last = plsc.load_gather(strikes_v, [zeros + (N_STR - 1)])
        k = jnp.clip(plsc.load_gather(strike_v, [zeros]), s_first, s_last)
        n_s = _count_le(strikes_v, N_STR, [256, 16, 1], k, iota)
        idx = jnp.clip(n_s - 1, 0, N_STR - 2)

        k_lo = plsc.load_gather(strikes_v, [idx])
        k_hi = plsc.load_gather(strikes_v, [idx + 1])
        w = jnp.clip((k - k_lo) / (k_hi - k_lo), 0.0, 1.0)

        r0 = jlo - rbase
        c0 = idx
        cp_w.wait()
        v00 = plsc.load_gather(win_v, [r0, c0])
        v01 = plsc.load_gather(win_v, [r0, c0 + 1])
        v10 = plsc.load_gather(win_v, [r0 + 1, c0])
        v11 = plsc.load_gather(win_v, [r0 + 1, c0 + 1])
        a0 = v00 + w * (v01 - v00)
        a1 = v10 + w * (v11 - v10)
        out_v[...] = a0 + u * (a1 - a0)
        pltpu.sync_copy(out_v.at[pl.ds(0, 1)], out_hbm)


def kernel(vols, strikes, expiries, strike, expiry):
    strike_v = strike.astype(jnp.float32).reshape(1)
    expiry_v = expiry.astype(jnp.float32).reshape(1)
    mesh = plsc.VectorSubcoreMesh(core_axis_name="c", subcore_axis_name="s",
                                  num_cores=1, num_subcores=1)
    run = pl.kernel(
        _body,
        out_type=jax.ShapeDtypeStruct((1,), jnp.float32),
        mesh=mesh,
        compiler_params=pltpu.CompilerParams(needs_layout_passes=False),
        scratch_types=[
            pltpu.VMEM((N_STR,), jnp.float32),
            pltpu.VMEM((N_EXP,), jnp.float32),
            pltpu.VMEM((1,), jnp.float32),
            pltpu.VMEM((1,), jnp.float32),
            pltpu.VMEM((ROWS, N_STR), jnp.float32),
            pltpu.VMEM((L,), jnp.float32),
            pltpu.SemaphoreType.DMA,
            pltpu.SemaphoreType.DMA,
            pltpu.SemaphoreType.DMA,
        ],
    )
    out = run(vols, strikes, expiries, strike_v, expiry_v)
    return out.reshape(())


# speedup vs baseline: 1.1049x; 1.1049x over previous
import jax
import jax.numpy as jnp
from jax import lax
from jax.experimental import pallas as pl
from jax.experimental.pallas import tpu as pltpu
from jax.experimental.pallas import tpu_sc as plsc

L = 16
N_EXP = 1024
N_STR = 4096
ROWS = 16
COLS = 256


def _probe_count(grid_v, base, st, x_vec, iota, n=None):
    probe = base + iota * st
    if n is None:
        vals = plsc.load_gather(grid_v, [probe])
        le = vals <= x_vec
    else:
        vals = plsc.load_gather(grid_v, [jnp.minimum(probe, n - 1)])
        le = jnp.logical_and(vals <= x_vec, probe < n)
    return plsc.all_reduce_population_count(le)


def _body(vols_hbm, strikes_hbm, expiries_hbm, strike_hbm, expiry_hbm,
          out_hbm, strikes_v, expiries_v, strike_v, expiry_v, win_v, out_v,
          sem_e, sem_s, sem_w):
    c = lax.axis_index("c")
    s = lax.axis_index("s")

    @pl.when(jnp.logical_and(c == 0, s == 0))
    def _():
        cp_e = pltpu.async_copy(expiries_hbm, expiries_v, sem_e)
        cp_t = pltpu.async_copy(expiry_hbm, expiry_v, sem_e)
        cp_s = pltpu.async_copy(strikes_hbm, strikes_v, sem_s)
        cp_k = pltpu.async_copy(strike_hbm, strike_v, sem_s)

        iota = lax.iota(jnp.int32, L)
        zeros = iota * 0

        cp_e.wait()
        cp_t.wait()
        t = plsc.load_gather(expiry_v, [zeros])
        c1 = _probe_count(expiries_v, zeros, 64, t, iota)
        b1 = jnp.maximum(c1 - 1, 0) * 64
        c2 = _probe_count(expiries_v, b1, 4, t, iota)
        b2 = b1 + jnp.maximum(c2 - 1, 0) * 4
        c3 = _probe_count(expiries_v, b2, 1, t, iota, n=N_EXP)
        n_e = b2 + c3
        jhi = jnp.clip(n_e, 1, N_EXP - 1)
        jlo = jhi - 1
        e_lo = plsc.load_gather(expiries_v, [jlo])
        e_hi = plsc.load_gather(expiries_v, [jhi])
        u = jnp.clip((t - e_lo) / (e_hi - e_lo), 0.0, 1.0)

        cp_s.wait()
        cp_k.wait()
        k = plsc.load_gather(strike_v, [zeros])
        d1 = _probe_count(strikes_v, zeros, 256, k, iota)
        a1 = jnp.maximum(d1 - 1, 0) * 256
        d2 = _probe_count(strikes_v, a1, 16, k, iota)
        a2 = a1 + jnp.maximum(d2 - 1, 0) * 16
        rbase = (jlo >> 3) << 3
        cbase = jnp.minimum((jnp.maximum(a2 - 1, 0) >> 7) << 7, N_STR - COLS)
        rbase_s = pl.multiple_of(jnp.max(rbase), 8)
        cbase_s = pl.multiple_of(jnp.max(cbase), 128)
        cp_w = pltpu.async_copy(
            vols_hbm.at[pl.ds(rbase_s, ROWS), pl.ds(cbase_s, COLS)],
            win_v, sem_w)

        d3 = _probe_count(strikes_v, a2, 1, k, iota)
        idx = jnp.clip(a2 + d3 - 1, 0, N_STR - 2)
        k_lo = plsc.load_gather(strikes_v, [idx])
        k_hi = plsc.load_gather(strikes_v, [idx + 1])
        w = jnp.clip((k - k_lo) / (k_hi - k_lo), 0.0, 1.0)

        r0 = jlo - rbase
        c0 = idx - cbase
        cp_w.wait()
        v00 = plsc.load_gather(win_v, [r0, c0])
        v01 = plsc.load_gather(win_v, [r0, c0 + 1])
        v10 = plsc.load_gather(win_v, [r0 + 1, c0])
        v11 = plsc.load_gather(win_v, [r0 + 1, c0 + 1])
        a_lo = v00 + w * (v01 - v00)
        a_hi = v10 + w * (v11 - v10)
        out_v[...] = a_lo + u * (a_hi - a_lo)
        pltpu.sync_copy(out_v.at[pl.ds(0, 1)], out_hbm)


def kernel(vols, strikes, expiries, strike, expiry):
    strike_v = strike.astype(jnp.float32).reshape(1)
    expiry_v = expiry.astype(jnp.float32).reshape(1)
    mesh = plsc.VectorSubcoreMesh(core_axis_name="c", subcore_axis_name="s",
                                  num_cores=1, num_subcores=1)
    run = pl.kernel(
        _body,
        out_type=jax.ShapeDtypeStruct((1,), jnp.float32),
        mesh=mesh,
        compiler_params=pltpu.CompilerParams(needs_layout_passes=False),
        scratch_types=[
            pltpu.VMEM((N_STR,), jnp.float32),
            pltpu.VMEM((N_EXP,), jnp.float32),
            pltpu.VMEM((1,), jnp.float32),
            pltpu.VMEM((1,), jnp.float32),
            pltpu.VMEM((ROWS, COLS), jnp.float32),
            pltpu.VMEM((L,), jnp.float32),
            pltpu.SemaphoreType.DMA,
            pltpu.SemaphoreType.DMA,
            pltpu.SemaphoreType.DMA,
        ],
    )
    out = run(vols, strikes, expiries, strike_v, expiry_v)
    return out.reshape(())
